# baseline (speedup 1.0000x reference)
import functools

import jax
import jax.numpy as jnp
from jax import lax
from jax.experimental import pallas as pl
from jax.experimental.pallas import tpu as pltpu
from jax.experimental.pallas import tpu_sc as plsc

_D = 256
_N = 8192
_T = 2304
_BETA = 0.25

_NB = 512
_TB = 768

_BIGF = 3e38
_BIGI = 2**30


def _scan_body(e_ref, xm2t_ref, iv_ref, mv_ref,
               sbuf_ref, esqbuf_ref, v1_ref, i1_ref):
    n = pl.program_id(1)
    nn = pl.num_programs(1)

    def epilogue(slot, blk):
        dist = (sbuf_ref[slot] + esqbuf_ref[slot]).reshape(_NB // 8, 8, _TB)
        row = (blk * _NB
               + lax.broadcasted_iota(jnp.int32, dist.shape, 0) * 8
               + lax.broadcasted_iota(jnp.int32, dist.shape, 1))
        m8 = jnp.min(dist, axis=0)
        id8 = jnp.min(jnp.where(dist == m8[None], row, _BIGI), axis=0)
        v1, i1 = v1_ref[...], i1_ref[...]
        better = m8 < v1
        v1_ref[...] = jnp.where(better, m8, v1)
        i1_ref[...] = jnp.where(better, id8, i1)

    @pl.when(n == 0)
    def _():
        v1_ref[...] = jnp.full((8, _TB), _BIGF, jnp.float32)
        i1_ref[...] = jnp.zeros((8, _TB), jnp.int32)

    @pl.when(n > 0)
    def _():
        epilogue((n - 1) % 2, n - 1)

    ef = e_ref[...]
    cur = n % 2
    sbuf_ref[cur] = jnp.dot(ef, xm2t_ref[...],
                            preferred_element_type=jnp.float32)
    esqbuf_ref[cur] = jnp.sum(ef * ef, axis=1, keepdims=True)

    @pl.when(n == nn - 1)
    def _():
        epilogue(n % 2, n)
        cval, cidx = v1_ref[...], i1_ref[...]
        m = jnp.min(cval, axis=0, keepdims=True)
        mv_ref[...] = m
        iv_ref[...] = jnp.min(jnp.where(cval == m, cidx, _BIGI),
                              axis=0, keepdims=True)


def _vq_scan(emb, xm2t):
    return pl.pallas_call(
        _scan_body,
        grid=(_T // _TB, _N // _NB),
        in_specs=[
            pl.BlockSpec((_NB, _D), lambda t, n: (n, 0)),
            pl.BlockSpec((_D, _TB), lambda t, n: (0, t)),
        ],
        out_specs=[
            pl.BlockSpec((1, _TB), lambda t, n: (0, t)),
            pl.BlockSpec((1, _TB), lambda t, n: (0, t)),
        ],
        out_shape=[
            jax.ShapeDtypeStruct((1, _T), jnp.int32),
            jax.ShapeDtypeStruct((1, _T), jnp.float32),
        ],
        scratch_shapes=[
            pltpu.VMEM((2, _NB, _TB), jnp.float32),
            pltpu.VMEM((2, _NB, 1), jnp.float32),
            pltpu.VMEM((8, _TB), jnp.float32),
            pltpu.VMEM((8, _TB), jnp.int32),
        ],
    )(emb, xm2t)


def _sc_gather(table, idx):
    info = plsc.get_sparse_core_info()
    nc, ns = info.num_cores, info.num_subcores
    nw = nc * ns
    chunk = _T // nw
    mesh = plsc.VectorSubcoreMesh(core_axis_name="c", subcore_axis_name="s")

    @functools.partial(
        pl.kernel,
        mesh=mesh,
        out_type=jax.ShapeDtypeStruct((_T, _D), jnp.float32),
        scratch_types=[
            pltpu.VMEM((chunk,), jnp.int32),
            pltpu.VMEM((chunk, _D), jnp.float32),
            pltpu.SemaphoreType.DMA,
        ],
    )
    def k(table_hbm, idx_hbm, out_hbm, idx_v, rows_v, sem):
        wid = lax.axis_index("s") * nc + lax.axis_index("c")
        base = wid * chunk
        pltpu.sync_copy(idx_hbm.at[pl.ds(base, chunk)], idx_v)
        pltpu.async_copy(table_hbm.at[idx_v], rows_v, sem).wait()
        pltpu.sync_copy(rows_v, out_hbm.at[pl.ds(base, chunk)])

    return k(table, idx)


def _mlp_body(x_ref, z_ref, mv_ref, w1_ref, b1_ref, w2_ref, b2_ref,
              zout_ref, loss_ref):
    t = pl.program_id(0)
    x = x_ref[...]
    z = z_ref[...]
    r = x - z
    h = jnp.maximum(
        jnp.dot(r, w1_ref[...], preferred_element_type=jnp.float32)
        + b1_ref[...], 0.0)
    zout_ref[...] = (z + jnp.dot(h, w2_ref[...],
                                 preferred_element_type=jnp.float32)
                     + b2_ref[...])
    psum = (jnp.sum(mv_ref[...], keepdims=True)
            + jnp.sum(x * x, keepdims=True))
    prev = jnp.where(t == 0, jnp.zeros_like(psum), loss_ref[...])
    tot = prev + psum
    nblk = pl.num_programs(0)
    loss_ref[...] = jnp.where(t == nblk - 1, tot * (_BETA / _T), tot)


def _mlp(x2, z2, mv, w1, b1, w2, b2):
    return pl.pallas_call(
        _mlp_body,
        grid=(_T // _TB,),
        in_specs=[
            pl.BlockSpec((_TB, _D), lambda t: (t, 0)),
            pl.BlockSpec((_TB, _D), lambda t: (t, 0)),
            pl.BlockSpec((1, _TB), lambda t: (0, t)),
            pl.BlockSpec((_D, _D), lambda t: (0, 0)),
            pl.BlockSpec((1, _D), lambda t: (0, 0)),
            pl.BlockSpec((_D, _D), lambda t: (0, 0)),
            pl.BlockSpec((1, _D), lambda t: (0, 0)),
        ],
        out_specs=[
            pl.BlockSpec((_TB, _D), lambda t: (t, 0)),
            pl.BlockSpec((1, 1), lambda t: (0, 0)),
        ],
        out_shape=[
            jax.ShapeDtypeStruct((_T, _D), jnp.float32),
            jax.ShapeDtypeStruct((1, 1), jnp.float32),
        ],
    )(x2, z2, mv, w1, b1, w2, b2)


def kernel(x, embedding, W1, b1, W2, b2):
    x2 = x.reshape(_T, _D)
    xm2t = (-2.0 * x2).T
    minidx, minval = _vq_scan(embedding, xm2t)
    z2 = _sc_gather(embedding, minidx.reshape(_T))
    zout, loss = _mlp(x2, z2, minval, W1, b1.reshape(1, _D),
                      W2, b2.reshape(1, _D))
    return zout.reshape(x.shape), loss[0, 0]


# speedup vs baseline: 1.0815x; 1.0815x over previous
import functools

import jax
import jax.numpy as jnp
from jax import lax
from jax.experimental import pallas as pl
from jax.experimental.pallas import tpu as pltpu
from jax.experimental.pallas import tpu_sc as plsc

_D = 256
_N = 8192
_T = 2304
_BETA = 0.25

_NB = 512
_TB = 768

_BIGF = 3e38
_BIGI = 2**30


def _scan_body(e_ref, xm2t_ref, iv_ref, mv_ref, v1_ref, i1_ref):
    n = pl.program_id(1)
    ef = e_ref[...]
    s = jnp.dot(ef, xm2t_ref[...], preferred_element_type=jnp.float32)
    esq = jnp.sum(ef * ef, axis=1, keepdims=True)
    dist = (s + esq).reshape(_NB // 8, 8, _TB)
    row = (n * _NB
           + lax.broadcasted_iota(jnp.int32, dist.shape, 0) * 8
           + lax.broadcasted_iota(jnp.int32, dist.shape, 1))
    m8 = jnp.min(dist, axis=0)
    id8 = jnp.min(jnp.where(dist == m8[None], row, _BIGI), axis=0)

    @pl.when(n == 0)
    def _():
        v1_ref[...] = m8
        i1_ref[...] = id8

    @pl.when(n > 0)
    def _():
        v1, i1 = v1_ref[...], i1_ref[...]
        better = m8 < v1
        v1_ref[...] = jnp.where(better, m8, v1)
        i1_ref[...] = jnp.where(better, id8, i1)

    @pl.when(n == pl.num_programs(1) - 1)
    def _():
        cval, cidx = v1_ref[...], i1_ref[...]
        m = jnp.min(cval, axis=0, keepdims=True)
        mv_ref[...] = m
        iv_ref[...] = jnp.min(jnp.where(cval == m, cidx, _BIGI),
                              axis=0, keepdims=True)


def _vq_scan(emb, xm2t):
    return pl.pallas_call(
        _scan_body,
        grid=(_T // _TB, _N // _NB),
        in_specs=[
            pl.BlockSpec((_NB, _D), lambda t, n: (n, 0)),
            pl.BlockSpec((_D, _TB), lambda t, n: (0, t)),
        ],
        out_specs=[
            pl.BlockSpec((1, _TB), lambda t, n: (0, t)),
            pl.BlockSpec((1, _TB), lambda t, n: (0, t)),
        ],
        out_shape=[
            jax.ShapeDtypeStruct((1, _T), jnp.int32),
            jax.ShapeDtypeStruct((1, _T), jnp.float32),
        ],
        scratch_shapes=[
            pltpu.VMEM((8, _TB), jnp.float32),
            pltpu.VMEM((8, _TB), jnp.int32),
        ],
    )(emb, xm2t)


def _sc_gather(table, idx):
    info = plsc.get_sparse_core_info()
    nc, ns = info.num_cores, info.num_subcores
    nw = nc * ns
    chunk = _T // nw
    mesh = plsc.VectorSubcoreMesh(core_axis_name="c", subcore_axis_name="s")

    @functools.partial(
        pl.kernel,
        mesh=mesh,
        out_type=jax.ShapeDtypeStruct((_T, _D), jnp.float32),
        scratch_types=[
            pltpu.VMEM((chunk,), jnp.int32),
            pltpu.VMEM((chunk, _D), jnp.float32),
            pltpu.SemaphoreType.DMA,
        ],
    )
    def k(table_hbm, idx_hbm, out_hbm, idx_v, rows_v, sem):
        wid = lax.axis_index("s") * nc + lax.axis_index("c")
        base = wid * chunk
        pltpu.sync_copy(idx_hbm.at[pl.ds(base, chunk)], idx_v)
        pltpu.async_copy(table_hbm.at[idx_v], rows_v, sem).wait()
        pltpu.sync_copy(rows_v, out_hbm.at[pl.ds(base, chunk)])

    return k(table, idx)


def _mlp_body(x_ref, z_ref, mv_ref, w1_ref, b1_ref, w2_ref, b2_ref,
              zout_ref, loss_ref):
    t = pl.program_id(0)
    x = x_ref[...]
    z = z_ref[...]
    r = x - z
    h = jnp.maximum(
        jnp.dot(r, w1_ref[...], preferred_element_type=jnp.float32)
        + b1_ref[...], 0.0)
    zout_ref[...] = (z + jnp.dot(h, w2_ref[...],
                                 preferred_element_type=jnp.float32)
                     + b2_ref[...])
    psum = (jnp.sum(mv_ref[...], keepdims=True)
            + jnp.sum(x * x, keepdims=True))
    prev = jnp.where(t == 0, jnp.zeros_like(psum), loss_ref[...])
    tot = prev + psum
    nblk = pl.num_programs(0)
    loss_ref[...] = jnp.where(t == nblk - 1, tot * (_BETA / _T), tot)


def _mlp(x2, z2, mv, w1, b1, w2, b2):
    return pl.pallas_call(
        _mlp_body,
        grid=(_T // _TB,),
        in_specs=[
            pl.BlockSpec((_TB, _D), lambda t: (t, 0)),
            pl.BlockSpec((_TB, _D), lambda t: (t, 0)),
            pl.BlockSpec((1, _TB), lambda t: (0, t)),
            pl.BlockSpec((_D, _D), lambda t: (0, 0)),
            pl.BlockSpec((1, _D), lambda t: (0, 0)),
            pl.BlockSpec((_D, _D), lambda t: (0, 0)),
            pl.BlockSpec((1, _D), lambda t: (0, 0)),
        ],
        out_specs=[
            pl.BlockSpec((_TB, _D), lambda t: (t, 0)),
            pl.BlockSpec((1, 1), lambda t: (0, 0)),
        ],
        out_shape=[
            jax.ShapeDtypeStruct((_T, _D), jnp.float32),
            jax.ShapeDtypeStruct((1, 1), jnp.float32),
        ],
    )(x2, z2, mv, w1, b1, w2, b2)


def kernel(x, embedding, W1, b1, W2, b2):
    x2 = x.reshape(_T, _D)
    xm2t = (-2.0 * x2).T
    minidx, minval = _vq_scan(embedding, xm2t)
    z2 = _sc_gather(embedding, minidx.reshape(_T))
    zout, loss = _mlp(x2, z2, minval, W1, b1.reshape(1, _D),
                      W2, b2.reshape(1, _D))
    return zout.reshape(x.shape), loss[0, 0]


# speedup vs baseline: 1.1393x; 1.0535x over previous
import functools

import jax
import jax.numpy as jnp
from jax import lax
from jax.experimental import pallas as pl
from jax.experimental.pallas import tpu as pltpu
from jax.experimental.pallas import tpu_sc as plsc

_D = 256
_N = 8192
_T = 2304
_BETA = 0.25

_NB = 512
_TB = 768

_BIGF = 3e38
_BIGI = 2**30


_SUB = 128


def _scan_body(e_ref, x_ref, iv_ref, mv_ref, xt_ref, v1_ref, i1_ref):
    n = pl.program_id(1)

    @pl.when(n == 0)
    def _():
        xt_ref[...] = x_ref[...].T * -2.0

    xt = xt_ref[...]
    m8 = None
    for sb in range(_NB // _SUB):
        ef = e_ref[pl.ds(sb * _SUB, _SUB), :]
        s = jnp.dot(ef, xt, preferred_element_type=jnp.float32)
        esq = jnp.sum(ef * ef, axis=1, keepdims=True)
        dist = (s + esq).reshape(_SUB // 8, 8, _TB)
        row = (n * _NB + sb * _SUB
               + lax.broadcasted_iota(jnp.int32, dist.shape, 0) * 8
               + lax.broadcasted_iota(jnp.int32, dist.shape, 1))
        msb = jnp.min(dist, axis=0)
        isb = jnp.min(jnp.where(dist == msb[None], row, _BIGI), axis=0)
        if m8 is None:
            m8, id8 = msb, isb
        else:
            upd = msb < m8
            m8 = jnp.where(upd, msb, m8)
            id8 = jnp.where(upd, isb, id8)

    @pl.when(n == 0)
    def _():
        v1_ref[...] = m8
        i1_ref[...] = id8

    @pl.when(n > 0)
    def _():
        v1, i1 = v1_ref[...], i1_ref[...]
        better = m8 < v1
        v1_ref[...] = jnp.where(better, m8, v1)
        i1_ref[...] = jnp.where(better, id8, i1)

    @pl.when(n == pl.num_programs(1) - 1)
    def _():
        cval, cidx = v1_ref[...], i1_ref[...]
        m = jnp.min(cval, axis=0, keepdims=True)
        mv_ref[...] = m
        iv_ref[...] = jnp.min(jnp.where(cval == m, cidx, _BIGI),
                              axis=0, keepdims=True)


def _vq_scan(emb, x2):
    return pl.pallas_call(
        _scan_body,
        grid=(_T // _TB, _N // _NB),
        in_specs=[
            pl.BlockSpec((_NB, _D), lambda t, n: (n, 0)),
            pl.BlockSpec((_TB, _D), lambda t, n: (t, 0)),
        ],
        out_specs=[
            pl.BlockSpec((1, _TB), lambda t, n: (0, t)),
            pl.BlockSpec((1, _TB), lambda t, n: (0, t)),
        ],
        out_shape=[
            jax.ShapeDtypeStruct((1, _T), jnp.int32),
            jax.ShapeDtypeStruct((1, _T), jnp.float32),
        ],
        scratch_shapes=[
            pltpu.VMEM((_D, _TB), jnp.float32),
            pltpu.VMEM((8, _TB), jnp.float32),
            pltpu.VMEM((8, _TB), jnp.int32),
        ],
    )(emb, x2)


def _sc_gather(table, idx):
    info = plsc.get_sparse_core_info()
    nc, ns = info.num_cores, info.num_subcores
    nw = nc * ns
    chunk = _T // nw
    mesh = plsc.VectorSubcoreMesh(core_axis_name="c", subcore_axis_name="s")

    @functools.partial(
        pl.kernel,
        mesh=mesh,
        out_type=jax.ShapeDtypeStruct((_T, _D), jnp.float32),
        scratch_types=[
            pltpu.VMEM((chunk,), jnp.int32),
            pltpu.VMEM((chunk, _D), jnp.float32),
            pltpu.SemaphoreType.DMA,
        ],
    )
    def k(table_hbm, idx_hbm, out_hbm, idx_v, rows_v, sem):
        wid = lax.axis_index("s") * nc + lax.axis_index("c")
        base = wid * chunk
        pltpu.sync_copy(idx_hbm.at[pl.ds(base, chunk)], idx_v)
        pltpu.async_copy(table_hbm.at[idx_v], rows_v, sem).wait()
        pltpu.sync_copy(rows_v, out_hbm.at[pl.ds(base, chunk)])

    return k(table, idx)


def _mlp_body(x_ref, z_ref, mv_ref, w1_ref, b1_ref, w2_ref, b2_ref,
              zout_ref, loss_ref):
    t = pl.program_id(0)
    x = x_ref[...]
    z = z_ref[...]
    r = x - z
    h = jnp.maximum(
        jnp.dot(r, w1_ref[...], preferred_element_type=jnp.float32)
        + b1_ref[...], 0.0)
    zout_ref[...] = (z + jnp.dot(h, w2_ref[...],
                                 preferred_element_type=jnp.float32)
                     + b2_ref[...])
    psum = (jnp.sum(mv_ref[...], keepdims=True)
            + jnp.sum(x * x, keepdims=True))
    prev = jnp.where(t == 0, jnp.zeros_like(psum), loss_ref[...])
    tot = prev + psum
    nblk = pl.num_programs(0)
    loss_ref[...] = jnp.where(t == nblk - 1, tot * (_BETA / _T), tot)


def _mlp(x2, z2, mv, w1, b1, w2, b2):
    return pl.pallas_call(
        _mlp_body,
        grid=(_T // _TB,),
        in_specs=[
            pl.BlockSpec((_TB, _D), lambda t: (t, 0)),
            pl.BlockSpec((_TB, _D), lambda t: (t, 0)),
            pl.BlockSpec((1, _TB), lambda t: (0, t)),
            pl.BlockSpec((_D, _D), lambda t: (0, 0)),
            pl.BlockSpec((1, _D), lambda t: (0, 0)),
            pl.BlockSpec((_D, _D), lambda t: (0, 0)),
            pl.BlockSpec((1, _D), lambda t: (0, 0)),
        ],
        out_specs=[
            pl.BlockSpec((_TB, _D), lambda t: (t, 0)),
            pl.BlockSpec((1, 1), lambda t: (0, 0)),
        ],
        out_shape=[
            jax.ShapeDtypeStruct((_T, _D), jnp.float32),
            jax.ShapeDtypeStruct((1, 1), jnp.float32),
        ],
    )(x2, z2, mv, w1, b1, w2, b2)


def kernel(x, embedding, W1, b1, W2, b2):
    x2 = x.reshape(_T, _D)
    minidx, minval = _vq_scan(embedding, x2)
    z2 = _sc_gather(embedding, minidx.reshape(_T))
    zout, loss = _mlp(x2, z2, minval, W1, b1.reshape(1, _D),
                      W2, b2.reshape(1, _D))
    return zout.reshape(x.shape), loss[0, 0]


# speedup vs baseline: 1.5365x; 1.3486x over previous
import functools

import jax
import jax.numpy as jnp
from jax import lax
from jax.experimental import pallas as pl
from jax.experimental.pallas import tpu as pltpu
from jax.experimental.pallas import tpu_sc as plsc

_D = 256
_N = 8192
_T = 2304
_BETA = 0.25

_NB = 512
_TB = 2304

_BIGF = 3e38
_BIGI = 2**30


_SUB = 128


def _scan_body(e_ref, x_ref, iv_ref, mv_ref, xt_ref, v1_ref, i1_ref):
    n = pl.program_id(1)

    @pl.when(n == 0)
    def _():
        xt_ref[...] = x_ref[...].T * -2.0

    xt = xt_ref[...]
    m8 = None
    for sb in range(_NB // _SUB):
        ef = e_ref[pl.ds(sb * _SUB, _SUB), :]
        s = jnp.dot(ef, xt, preferred_element_type=jnp.float32)
        esq = jnp.sum(ef * ef, axis=1, keepdims=True)
        dist = (s + esq).reshape(_SUB // 8, 8, _TB)
        row = (n * _NB + sb * _SUB
               + lax.broadcasted_iota(jnp.int32, dist.shape, 0) * 8
               + lax.broadcasted_iota(jnp.int32, dist.shape, 1))
        msb = jnp.min(dist, axis=0)
        isb = jnp.min(jnp.where(dist == msb[None], row, _BIGI), axis=0)
        if m8 is None:
            m8, id8 = msb, isb
        else:
            upd = msb < m8
            m8 = jnp.where(upd, msb, m8)
            id8 = jnp.where(upd, isb, id8)

    @pl.when(n == 0)
    def _():
        v1_ref[...] = m8
        i1_ref[...] = id8

    @pl.when(n > 0)
    def _():
        v1, i1 = v1_ref[...], i1_ref[...]
        better = m8 < v1
        v1_ref[...] = jnp.where(better, m8, v1)
        i1_ref[...] = jnp.where(better, id8, i1)

    @pl.when(n == pl.num_programs(1) - 1)
    def _():
        cval, cidx = v1_ref[...], i1_ref[...]
        m = jnp.min(cval, axis=0, keepdims=True)
        mv_ref[...] = m
        iv_ref[...] = jnp.min(jnp.where(cval == m, cidx, _BIGI),
                              axis=0, keepdims=True)


def _vq_scan(emb, x2):
    return pl.pallas_call(
        _scan_body,
        grid=(_T // _TB, _N // _NB),
        in_specs=[
            pl.BlockSpec((_NB, _D), lambda t, n: (n, 0)),
            pl.BlockSpec((_TB, _D), lambda t, n: (t, 0)),
        ],
        out_specs=[
            pl.BlockSpec((1, _TB), lambda t, n: (0, t)),
            pl.BlockSpec((1, _TB), lambda t, n: (0, t)),
        ],
        out_shape=[
            jax.ShapeDtypeStruct((1, _T), jnp.int32),
            jax.ShapeDtypeStruct((1, _T), jnp.float32),
        ],
        scratch_shapes=[
            pltpu.VMEM((_D, _TB), jnp.float32),
            pltpu.VMEM((8, _TB), jnp.float32),
            pltpu.VMEM((8, _TB), jnp.int32),
        ],
    )(emb, x2)


def _sc_gather(table, idx):
    info = plsc.get_sparse_core_info()
    nc, ns = info.num_cores, info.num_subcores
    nw = nc * ns
    chunk = _T // nw
    mesh = plsc.VectorSubcoreMesh(core_axis_name="c", subcore_axis_name="s")

    @functools.partial(
        pl.kernel,
        mesh=mesh,
        out_type=jax.ShapeDtypeStruct((_T, _D), jnp.float32),
        scratch_types=[
            pltpu.VMEM((chunk,), jnp.int32),
            pltpu.VMEM((chunk, _D), jnp.float32),
            pltpu.SemaphoreType.DMA,
        ],
    )
    def k(table_hbm, idx_hbm, out_hbm, idx_v, rows_v, sem):
        wid = lax.axis_index("s") * nc + lax.axis_index("c")
        base = wid * chunk
        pltpu.sync_copy(idx_hbm.at[pl.ds(base, chunk)], idx_v)
        pltpu.async_copy(table_hbm.at[idx_v], rows_v, sem).wait()
        pltpu.sync_copy(rows_v, out_hbm.at[pl.ds(base, chunk)])

    return k(table, idx)


def _mlp_body(x_ref, z_ref, mv_ref, w1_ref, b1_ref, w2_ref, b2_ref,
              zout_ref, loss_ref):
    t = pl.program_id(0)
    x = x_ref[...]
    z = z_ref[...]
    r = x - z
    h = jnp.maximum(
        jnp.dot(r, w1_ref[...], preferred_element_type=jnp.float32)
        + b1_ref[...], 0.0)
    zout_ref[...] = (z + jnp.dot(h, w2_ref[...],
                                 preferred_element_type=jnp.float32)
                     + b2_ref[...])
    psum = (jnp.sum(mv_ref[...], keepdims=True)
            + jnp.sum(x * x, keepdims=True))
    prev = jnp.where(t == 0, jnp.zeros_like(psum), loss_ref[...])
    tot = prev + psum
    nblk = pl.num_programs(0)
    loss_ref[...] = jnp.where(t == nblk - 1, tot * (_BETA / _T), tot)


def _mlp(x2, z2, mv, w1, b1, w2, b2):
    return pl.pallas_call(
        _mlp_body,
        grid=(_T // _TB,),
        in_specs=[
            pl.BlockSpec((_TB, _D), lambda t: (t, 0)),
            pl.BlockSpec((_TB, _D), lambda t: (t, 0)),
            pl.BlockSpec((1, _TB), lambda t: (0, t)),
            pl.BlockSpec((_D, _D), lambda t: (0, 0)),
            pl.BlockSpec((1, _D), lambda t: (0, 0)),
            pl.BlockSpec((_D, _D), lambda t: (0, 0)),
            pl.BlockSpec((1, _D), lambda t: (0, 0)),
        ],
        out_specs=[
            pl.BlockSpec((_TB, _D), lambda t: (t, 0)),
            pl.BlockSpec((1, 1), lambda t: (0, 0)),
        ],
        out_shape=[
            jax.ShapeDtypeStruct((_T, _D), jnp.float32),
            jax.ShapeDtypeStruct((1, 1), jnp.float32),
        ],
    )(x2, z2, mv, w1, b1, w2, b2)


def kernel(x, embedding, W1, b1, W2, b2):
    x2 = x.reshape(_T, _D)
    minidx, minval = _vq_scan(embedding, x2)
    z2 = _sc_gather(embedding, minidx.reshape(_T))
    zout, loss = _mlp(x2, z2, minval, W1, b1.reshape(1, _D),
                      W2, b2.reshape(1, _D))
    return zout.reshape(x.shape), loss[0, 0]


# speedup vs baseline: 1.5391x; 1.0017x over previous
import functools

import jax
import jax.numpy as jnp
from jax import lax
from jax.experimental import pallas as pl
from jax.experimental.pallas import tpu as pltpu
from jax.experimental.pallas import tpu_sc as plsc

_D = 256
_N = 8192
_T = 2304
_BETA = 0.25

_NB = 512
_TB = 2304

_BIGF = 3e38
_BIGI = 2**30


_SUB = 128


def _scan_body(e_ref, x_ref, iv_ref, mv_ref, xt_ref, v1_ref, i1_ref):
    n = pl.program_id(1)

    @pl.when(n == 0)
    def _():
        xt_ref[...] = x_ref[...].T * -2.0

    xt = xt_ref[...]
    m8 = None
    for sb in range(_NB // _SUB):
        ef = e_ref[pl.ds(sb * _SUB, _SUB), :]
        s = jnp.dot(ef, xt, preferred_element_type=jnp.float32)
        esq = jnp.sum(ef * ef, axis=1, keepdims=True)
        dist = (s + esq).reshape(_SUB // 8, 8, _TB)
        row = (n * _NB + sb * _SUB
               + lax.broadcasted_iota(jnp.int32, dist.shape, 0) * 8
               + lax.broadcasted_iota(jnp.int32, dist.shape, 1))
        msb = jnp.min(dist, axis=0)
        isb = jnp.min(jnp.where(dist == msb[None], row, _BIGI), axis=0)
        if m8 is None:
            m8, id8 = msb, isb
        else:
            upd = msb < m8
            m8 = jnp.where(upd, msb, m8)
            id8 = jnp.where(upd, isb, id8)

    @pl.when(n == 0)
    def _():
        v1_ref[...] = m8
        i1_ref[...] = id8

    @pl.when(n > 0)
    def _():
        v1, i1 = v1_ref[...], i1_ref[...]
        better = m8 < v1
        v1_ref[...] = jnp.where(better, m8, v1)
        i1_ref[...] = jnp.where(better, id8, i1)

    @pl.when(n == pl.num_programs(1) - 1)
    def _():
        cval, cidx = v1_ref[...], i1_ref[...]
        m = jnp.min(cval, axis=0, keepdims=True)
        mv_ref[...] = m
        iv_ref[...] = jnp.min(jnp.where(cval == m, cidx, _BIGI),
                              axis=0, keepdims=True)


def _vq_scan(emb, x2):
    return pl.pallas_call(
        _scan_body,
        grid=(_T // _TB, _N // _NB),
        in_specs=[
            pl.BlockSpec((_NB, _D), lambda t, n: (n, 0)),
            pl.BlockSpec((_TB, _D), lambda t, n: (t, 0)),
        ],
        out_specs=[
            pl.BlockSpec((1, _TB), lambda t, n: (0, t)),
            pl.BlockSpec((1, _TB), lambda t, n: (0, t)),
        ],
        out_shape=[
            jax.ShapeDtypeStruct((1, _T), jnp.int32),
            jax.ShapeDtypeStruct((1, _T), jnp.float32),
        ],
        scratch_shapes=[
            pltpu.VMEM((_D, _TB), jnp.float32),
            pltpu.VMEM((8, _TB), jnp.float32),
            pltpu.VMEM((8, _TB), jnp.int32),
        ],
    )(emb, x2)


def _sc_gather(table, idx):
    info = plsc.get_sparse_core_info()
    nc, ns = info.num_cores, info.num_subcores
    nw = nc * ns
    chunk = _T // nw
    mesh = plsc.VectorSubcoreMesh(core_axis_name="c", subcore_axis_name="s")

    h0 = 40
    h1 = chunk - h0

    @functools.partial(
        pl.kernel,
        mesh=mesh,
        out_type=jax.ShapeDtypeStruct((_T, _D), jnp.float32),
        scratch_types=[
            pltpu.VMEM((chunk,), jnp.int32),
            pltpu.VMEM((chunk, _D), jnp.float32),
            pltpu.SemaphoreType.DMA,
            pltpu.SemaphoreType.DMA,
        ],
    )
    def k(table_hbm, idx_hbm, out_hbm, idx_v, rows_v, sem_g, sem_o):
        wid = lax.axis_index("s") * nc + lax.axis_index("c")
        base = wid * chunk
        pltpu.sync_copy(idx_hbm.at[pl.ds(base, chunk)], idx_v)
        r0 = rows_v.at[pl.ds(0, h0)]
        r1 = rows_v.at[pl.ds(h0, h1)]
        g0 = pltpu.async_copy(table_hbm.at[idx_v.at[pl.ds(0, h0)]], r0, sem_g)
        g1 = pltpu.async_copy(table_hbm.at[idx_v.at[pl.ds(h0, h1)]], r1, sem_g)
        g0.wait()
        o0 = pltpu.async_copy(r0, out_hbm.at[pl.ds(base, h0)], sem_o)
        g1.wait()
        o1 = pltpu.async_copy(r1, out_hbm.at[pl.ds(base + h0, h1)], sem_o)
        o0.wait()
        o1.wait()

    return k(table, idx)


def _mlp_body(x_ref, z_ref, mv_ref, w1_ref, b1_ref, w2_ref, b2_ref,
              zout_ref, loss_ref):
    t = pl.program_id(0)
    x = x_ref[...]
    z = z_ref[...]
    r = x - z
    h = jnp.maximum(
        jnp.dot(r, w1_ref[...], preferred_element_type=jnp.float32)
        + b1_ref[...], 0.0)
    zout_ref[...] = (z + jnp.dot(h, w2_ref[...],
                                 preferred_element_type=jnp.float32)
                     + b2_ref[...])
    psum = (jnp.sum(mv_ref[...], keepdims=True)
            + jnp.sum(x * x, keepdims=True))
    prev = jnp.where(t == 0, jnp.zeros_like(psum), loss_ref[...])
    tot = prev + psum
    nblk = pl.num_programs(0)
    loss_ref[...] = jnp.where(t == nblk - 1, tot * (_BETA / _T), tot)


def _mlp(x2, z2, mv, w1, b1, w2, b2):
    return pl.pallas_call(
        _mlp_body,
        grid=(_T // _TB,),
        in_specs=[
            pl.BlockSpec((_TB, _D), lambda t: (t, 0)),
            pl.BlockSpec((_TB, _D), lambda t: (t, 0)),
            pl.BlockSpec((1, _TB), lambda t: (0, t)),
            pl.BlockSpec((_D, _D), lambda t: (0, 0)),
            pl.BlockSpec((1, _D), lambda t: (0, 0)),
            pl.BlockSpec((_D, _D), lambda t: (0, 0)),
            pl.BlockSpec((1, _D), lambda t: (0, 0)),
        ],
        out_specs=[
            pl.BlockSpec((_TB, _D), lambda t: (t, 0)),
            pl.BlockSpec((1, 1), lambda t: (0, 0)),
        ],
        out_shape=[
            jax.ShapeDtypeStruct((_T, _D), jnp.float32),
            jax.ShapeDtypeStruct((1, 1), jnp.float32),
        ],
    )(x2, z2, mv, w1, b1, w2, b2)


def kernel(x, embedding, W1, b1, W2, b2):
    x2 = x.reshape(_T, _D)
    minidx, minval = _vq_scan(embedding, x2)
    z2 = _sc_gather(embedding, minidx.reshape(_T))
    zout, loss = _mlp(x2, z2, minval, W1, b1.reshape(1, _D),
                      W2, b2.reshape(1, _D))
    return zout.reshape(x.shape), loss[0, 0]
